# baseline (speedup 1.0000x reference)
import functools

import jax
import jax.numpy as jnp
from jax import lax
from jax.experimental import pallas as pl
from jax.experimental.pallas import tpu as pltpu
from jax.experimental.pallas import tpu_sc as plsc

E = 8
TOPK = 2
D = 1024
F = 4096
T = 2048

BLK = 256
NP = TOPK * T + E * BLK
NB = NP // BLK
FT = 512
NF = F // FT

NC = 2
NS = 16
NW = NC * NS
CH = T // NW
RWL = 128

_INTERPRET = False


def _k1_body(x_ref, wr_ref, pos0_ref, pos1_ref, w0_ref, w1_ref, meta_ref):
    x = x_ref[...]
    xb = x.astype(jnp.bfloat16).astype(jnp.float32)
    logits = jnp.dot(xb, wr_ref[...], preferred_element_type=jnp.float32)
    m = jnp.max(logits, axis=1, keepdims=True)
    ex = jnp.exp(logits - m)
    p = ex / jnp.sum(ex, axis=1, keepdims=True)

    lane = lax.broadcasted_iota(jnp.int32, (T, E), 1)
    v0 = jnp.max(p, axis=1, keepdims=True)
    i0 = jnp.min(jnp.where(p == v0, lane, E), axis=1, keepdims=True)
    p1 = jnp.where(lane == i0, -jnp.inf, p)
    v1 = jnp.max(p1, axis=1, keepdims=True)
    i1 = jnp.min(jnp.where(p1 == v1, lane, E), axis=1, keepdims=True)
    s = v0 + v1
    w0n = v0 / s
    w1n = v1 / s

    a = (lane == i0).astype(jnp.int32) + (lane == i1).astype(jnp.int32)
    c = a
    sft = 1
    while sft < T:
        shifted = jnp.concatenate(
            [jnp.zeros((sft, E), jnp.int32), c[:T - sft]], axis=0)
        c = c + shifted
        sft *= 2
    p_exc = c - a
    counts = c[T - 1:T, :]
    padded = ((counts + (BLK - 1)) >> 8) << 8

    l8 = lax.broadcasted_iota(jnp.int32, (1, E), 1)
    st = padded
    sft = 1
    while sft < E:
        st = st + jnp.where(l8 >= sft, pltpu.roll(st, sft, axis=1), 0)
        sft *= 2
    starts = st - padded

    tot_f = (starts + p_exc).astype(jnp.float32)
    posv0 = jnp.sum(jnp.where(lane == i0, tot_f, 0.0), axis=1, keepdims=True)
    posv1 = jnp.sum(jnp.where(lane == i1, tot_f, 0.0), axis=1, keepdims=True)

    A = (jnp.where(lane == 0, posv0, 0.0) + jnp.where(lane == 1, posv1, 0.0)
         + jnp.where(lane == 2, w0n, 0.0) + jnp.where(lane == 3, w1n, 0.0))
    ii = lax.broadcasted_iota(jnp.int32, (E, E), 0)
    jj = lax.broadcasted_iota(jnp.int32, (E, E), 1)
    i8 = (ii == jj).astype(jnp.float32)
    at = lax.dot_general(i8, A, (((1,), (1,)), ((), ())),
                         preferred_element_type=jnp.float32,
                         precision=lax.Precision.HIGHEST)
    pos0_ref[...] = at[0:1].reshape(T).astype(jnp.int32)
    pos1_ref[...] = at[1:2].reshape(T).astype(jnp.int32)
    w0_ref[...] = jnp.broadcast_to(w0n, (T, RWL))
    w1_ref[...] = jnp.broadcast_to(w1n, (T, RWL))

    nactb = jnp.sum(padded) >> 8
    le = jnp.max(jnp.where(padded > 0, l8, -1))
    startsT = lax.dot_general(i8, starts.astype(jnp.float32),
                              (((1,), (1,)), ((), ())),
                              preferred_element_type=jnp.float32,
                              precision=lax.Precision.HIGHEST)
    paddedT = lax.dot_general(i8, padded.astype(jnp.float32),
                              (((1,), (1,)), ((), ())),
                              preferred_element_type=jnp.float32,
                              precision=lax.Precision.HIGHEST)
    bB = lax.broadcasted_iota(jnp.int32, (E, 128), 1).astype(
        jnp.float32) * float(BLK)
    in_e = jnp.logical_and(bB >= startsT, bB < startsT + paddedT)
    rowe = lax.broadcasted_iota(jnp.int32, (E, 128), 0).astype(jnp.float32)
    be = jnp.sum(rowe * in_e.astype(jnp.float32), axis=0,
                 keepdims=True).astype(jnp.int32)
    bi = lax.broadcasted_iota(jnp.int32, (1, 128), 1)
    meta = jnp.where(bi == 100, nactb, jnp.where(bi < nactb, be, le))
    meta_ref[...] = meta.reshape(128)


def _router_dispatch(x2d, w_router):
    return pl.pallas_call(
        _k1_body,
        out_shape=[
            jax.ShapeDtypeStruct((T,), jnp.int32),
            jax.ShapeDtypeStruct((T,), jnp.int32),
            jax.ShapeDtypeStruct((T, RWL), jnp.float32),
            jax.ShapeDtypeStruct((T, RWL), jnp.float32),
            jax.ShapeDtypeStruct((128,), jnp.int32),
        ],
        interpret=_INTERPRET,
    )(x2d, w_router)


def _k2_body(x_hbm, pos0_hbm, pos1_hbm, xs_hbm, idx0_v, idx1_v, xrows_v, sem):
    wid = lax.axis_index("s") * NC + lax.axis_index("c")
    base = wid * CH
    pltpu.sync_copy(pos0_hbm.at[pl.ds(base, CH)], idx0_v)
    pltpu.sync_copy(pos1_hbm.at[pl.ds(base, CH)], idx1_v)
    pltpu.sync_copy(x_hbm.at[pl.ds(base, CH)], xrows_v)
    pltpu.async_copy(xrows_v, xs_hbm.at[idx0_v], sem).wait()
    pltpu.async_copy(xrows_v, xs_hbm.at[idx1_v], sem).wait()


def _sc_scatter(x2d, pos0, pos1):
    mesh = plsc.VectorSubcoreMesh(core_axis_name="c", subcore_axis_name="s",
                                  num_cores=NC, num_subcores=NS)
    fn = functools.partial(
        pl.kernel,
        out_type=jax.ShapeDtypeStruct((NP, D), jnp.float32),
        mesh=mesh,
        scratch_types=[
            pltpu.VMEM((CH,), jnp.int32),
            pltpu.VMEM((CH,), jnp.int32),
            pltpu.VMEM((CH, D), jnp.float32),
            pltpu.SemaphoreType.DMA,
        ],
        interpret=_INTERPRET,
    )(_k2_body)
    return fn(x2d, pos0, pos1)


def _k3_body(s_ref, xs_ref, w1_ref, w3_ref, w2_ref, ys_ref, acc_ref):
    j = pl.program_id(0)
    b = pl.program_id(1)
    nact = s_ref[100]

    @pl.when(b < nact)
    def _():
        sl = pl.ds(b * BLK, BLK)
        x = xs_ref[sl, :]
        h1 = jnp.dot(x, w1_ref[0], preferred_element_type=jnp.float32)
        h3 = jnp.dot(x, w3_ref[0], preferred_element_type=jnp.float32)
        h = h1 * (1.0 / (1.0 + jnp.exp(-h1))) * h3
        part = jnp.dot(h, w2_ref[0], preferred_element_type=jnp.float32)

        @pl.when(j == 0)
        def _():
            acc_ref[sl, :] = part.astype(jnp.bfloat16)

        @pl.when(jnp.logical_and(j > 0, j < NF - 1))
        def _():
            acc_ref[sl, :] = (acc_ref[sl, :].astype(jnp.float32)
                              + part).astype(jnp.bfloat16)

        @pl.when(j == NF - 1)
        def _():
            ys_ref[...] = acc_ref[sl, :].astype(jnp.float32) + part


def _grouped_ffn(meta, xs, w1, w3, w2):
    grid_spec = pltpu.PrefetchScalarGridSpec(
        num_scalar_prefetch=1,
        grid=(NF, NB),
        in_specs=[
            pl.BlockSpec((NP, D), lambda j, b, s: (0, 0)),
            pl.BlockSpec((1, D, FT), lambda j, b, s: (s[b], 0, j)),
            pl.BlockSpec((1, D, FT), lambda j, b, s: (s[b], 0, j)),
            pl.BlockSpec((1, FT, D), lambda j, b, s: (s[b], j, 0)),
        ],
        out_specs=pl.BlockSpec(
            (BLK, D), lambda j, b, s: (jnp.where(j == NF - 1, b, 0), 0)),
        scratch_shapes=[pltpu.VMEM((NP, D), jnp.bfloat16)],
    )
    return pl.pallas_call(
        _k3_body,
        grid_spec=grid_spec,
        out_shape=jax.ShapeDtypeStruct((NP, D), jnp.float32),
        interpret=_INTERPRET,
    )(meta, xs, w1, w3, w2)


def _k4_body(ys_hbm, pos0_hbm, pos1_hbm, w0_hbm, w1_hbm, out_hbm,
             idxa_v, idxb_v, bufa_v, bufb_v, wa_v, wb_v, sem):
    wid = lax.axis_index("s") * NC + lax.axis_index("c")
    hh = CH // 2
    for h in range(2):
        base = wid * CH + h * hh
        pltpu.sync_copy(pos0_hbm.at[pl.ds(base, hh)], idxa_v)
        pltpu.sync_copy(pos1_hbm.at[pl.ds(base, hh)], idxb_v)
        pltpu.sync_copy(w0_hbm.at[pl.ds(base, hh)], wa_v)
        pltpu.sync_copy(w1_hbm.at[pl.ds(base, hh)], wb_v)
        pltpu.async_copy(ys_hbm.at[idxa_v], bufa_v, sem).wait()
        pltpu.async_copy(ys_hbm.at[idxb_v], bufb_v, sem).wait()

        def addrow(r, carry):
            w0v = wa_v[r, 0:16]
            w1v = wb_v[r, 0:16]
            for cblk in range(D // 16):
                cs = pl.ds(cblk * 16, 16)
                bufa_v[r, cs] = bufa_v[r, cs] * w0v + bufb_v[r, cs] * w1v
            return carry

        lax.fori_loop(0, hh, addrow, 0)
        pltpu.sync_copy(bufa_v, out_hbm.at[pl.ds(base, hh)])


def _sc_combine(ys, pos0, pos1, w0o, w1o):
    mesh = plsc.VectorSubcoreMesh(core_axis_name="c", subcore_axis_name="s",
                                  num_cores=NC, num_subcores=NS)
    fn = functools.partial(
        pl.kernel,
        out_type=jax.ShapeDtypeStruct((T, D), jnp.float32),
        mesh=mesh,
        scratch_types=[
            pltpu.VMEM((CH // 2,), jnp.int32),
            pltpu.VMEM((CH // 2,), jnp.int32),
            pltpu.VMEM((CH // 2, D), jnp.float32),
            pltpu.VMEM((CH // 2, D), jnp.float32),
            pltpu.VMEM((CH // 2, RWL), jnp.float32),
            pltpu.VMEM((CH // 2, RWL), jnp.float32),
            pltpu.SemaphoreType.DMA,
        ],
        interpret=_INTERPRET,
    )(_k4_body)
    return fn(ys, pos0, pos1, w0o, w1o)


def kernel(hidden_states, w_router, w1, w2, w3):
    shape = hidden_states.shape
    x2d = hidden_states.reshape(T, D)
    pos0, pos1, w0o, w1o, meta = _router_dispatch(x2d, w_router)
    xs = _sc_scatter(x2d, pos0, pos1)
    ys = _grouped_ffn(meta, xs, w1, w3, w2)
    out2d = _sc_combine(ys, pos0, pos1, w0o, w1o)
    return (out2d.reshape(shape), jnp.zeros((1,), jnp.float32))


# speedup vs baseline: 1.1406x; 1.1406x over previous
import functools

import jax
import jax.numpy as jnp
from jax import lax
from jax.experimental import pallas as pl
from jax.experimental.pallas import tpu as pltpu
from jax.experimental.pallas import tpu_sc as plsc

E = 8
TOPK = 2
D = 1024
F = 4096
T = 2048

BLK = 256
NP = TOPK * T + E * BLK
NB = NP // BLK
FT = 1024
NF = F // FT

NC = 2
NS = 16
NW = NC * NS
CH = T // NW
RWL = 128

_INTERPRET = False


def _k1_body(x_ref, wr_ref, pos0_ref, pos1_ref, w0_ref, w1_ref, meta_ref):
    x = x_ref[...]
    xb = x.astype(jnp.bfloat16).astype(jnp.float32)
    logits = jnp.dot(xb, wr_ref[...], preferred_element_type=jnp.float32)
    m = jnp.max(logits, axis=1, keepdims=True)
    ex = jnp.exp(logits - m)
    p = ex / jnp.sum(ex, axis=1, keepdims=True)

    lane = lax.broadcasted_iota(jnp.int32, (T, E), 1)
    v0 = jnp.max(p, axis=1, keepdims=True)
    i0 = jnp.min(jnp.where(p == v0, lane, E), axis=1, keepdims=True)
    p1 = jnp.where(lane == i0, -jnp.inf, p)
    v1 = jnp.max(p1, axis=1, keepdims=True)
    i1 = jnp.min(jnp.where(p1 == v1, lane, E), axis=1, keepdims=True)
    s = v0 + v1
    w0n = v0 / s
    w1n = v1 / s

    a = (lane == i0).astype(jnp.int32) + (lane == i1).astype(jnp.int32)
    c = a
    sft = 1
    while sft < T:
        shifted = jnp.concatenate(
            [jnp.zeros((sft, E), jnp.int32), c[:T - sft]], axis=0)
        c = c + shifted
        sft *= 2
    p_exc = c - a
    counts = c[T - 1:T, :]
    padded = ((counts + (BLK - 1)) >> 8) << 8

    l8 = lax.broadcasted_iota(jnp.int32, (1, E), 1)
    st = padded
    sft = 1
    while sft < E:
        st = st + jnp.where(l8 >= sft, pltpu.roll(st, sft, axis=1), 0)
        sft *= 2
    starts = st - padded

    tot_f = (starts + p_exc).astype(jnp.float32)
    posv0 = jnp.sum(jnp.where(lane == i0, tot_f, 0.0), axis=1, keepdims=True)
    posv1 = jnp.sum(jnp.where(lane == i1, tot_f, 0.0), axis=1, keepdims=True)

    A = (jnp.where(lane == 0, posv0, 0.0) + jnp.where(lane == 1, posv1, 0.0)
         + jnp.where(lane == 2, w0n, 0.0) + jnp.where(lane == 3, w1n, 0.0))
    ii = lax.broadcasted_iota(jnp.int32, (E, E), 0)
    jj = lax.broadcasted_iota(jnp.int32, (E, E), 1)
    i8 = (ii == jj).astype(jnp.float32)
    at = lax.dot_general(i8, A, (((1,), (1,)), ((), ())),
                         preferred_element_type=jnp.float32,
                         precision=lax.Precision.HIGHEST)
    pos0_ref[...] = at[0:1].reshape(T).astype(jnp.int32)
    pos1_ref[...] = at[1:2].reshape(T).astype(jnp.int32)
    w0_ref[...] = jnp.broadcast_to(w0n, (T, RWL))
    w1_ref[...] = jnp.broadcast_to(w1n, (T, RWL))

    nactb = jnp.sum(padded) >> 8
    le = jnp.max(jnp.where(padded > 0, l8, -1))
    startsT = lax.dot_general(i8, starts.astype(jnp.float32),
                              (((1,), (1,)), ((), ())),
                              preferred_element_type=jnp.float32,
                              precision=lax.Precision.HIGHEST)
    paddedT = lax.dot_general(i8, padded.astype(jnp.float32),
                              (((1,), (1,)), ((), ())),
                              preferred_element_type=jnp.float32,
                              precision=lax.Precision.HIGHEST)
    bB = lax.broadcasted_iota(jnp.int32, (E, 128), 1).astype(
        jnp.float32) * float(BLK)
    in_e = jnp.logical_and(bB >= startsT, bB < startsT + paddedT)
    rowe = lax.broadcasted_iota(jnp.int32, (E, 128), 0).astype(jnp.float32)
    be = jnp.sum(rowe * in_e.astype(jnp.float32), axis=0,
                 keepdims=True).astype(jnp.int32)
    bi = lax.broadcasted_iota(jnp.int32, (1, 128), 1)
    meta = jnp.where(bi == 100, nactb, jnp.where(bi < nactb, be, le))
    meta_ref[...] = meta.reshape(128)


def _router_dispatch(x2d, w_router):
    return pl.pallas_call(
        _k1_body,
        out_shape=[
            jax.ShapeDtypeStruct((T,), jnp.int32),
            jax.ShapeDtypeStruct((T,), jnp.int32),
            jax.ShapeDtypeStruct((T, RWL), jnp.float32),
            jax.ShapeDtypeStruct((T, RWL), jnp.float32),
            jax.ShapeDtypeStruct((128,), jnp.int32),
        ],
        interpret=_INTERPRET,
    )(x2d, w_router)


def _k2_body(x_hbm, pos0_hbm, pos1_hbm, xs_hbm, idx0_v, idx1_v, xrows_v, sem):
    wid = lax.axis_index("s") * NC + lax.axis_index("c")
    base = wid * CH
    pltpu.sync_copy(pos0_hbm.at[pl.ds(base, CH)], idx0_v)
    pltpu.sync_copy(pos1_hbm.at[pl.ds(base, CH)], idx1_v)
    pltpu.sync_copy(x_hbm.at[pl.ds(base, CH)], xrows_v)
    pltpu.async_copy(xrows_v, xs_hbm.at[idx0_v], sem).wait()
    pltpu.async_copy(xrows_v, xs_hbm.at[idx1_v], sem).wait()


def _sc_scatter(x2d, pos0, pos1):
    mesh = plsc.VectorSubcoreMesh(core_axis_name="c", subcore_axis_name="s",
                                  num_cores=NC, num_subcores=NS)
    fn = functools.partial(
        pl.kernel,
        out_type=jax.ShapeDtypeStruct((NP, D), jnp.float32),
        mesh=mesh,
        scratch_types=[
            pltpu.VMEM((CH,), jnp.int32),
            pltpu.VMEM((CH,), jnp.int32),
            pltpu.VMEM((CH, D), jnp.float32),
            pltpu.SemaphoreType.DMA,
        ],
        interpret=_INTERPRET,
    )(_k2_body)
    return fn(x2d, pos0, pos1)


def _k3_body(s_ref, xs_ref, w1_ref, w3_ref, w2_ref, ys_ref, acc_ref):
    j = pl.program_id(0)
    b = pl.program_id(1)
    nact = s_ref[100]

    @pl.when(b < nact)
    def _():
        sl = pl.ds(b * BLK, BLK)
        x = xs_ref[...]
        h1 = jnp.dot(x, w1_ref[0], preferred_element_type=jnp.float32)
        h3 = jnp.dot(x, w3_ref[0], preferred_element_type=jnp.float32)
        h = h1 * (1.0 / (1.0 + jnp.exp(-h1))) * h3
        part = jnp.dot(h, w2_ref[0], preferred_element_type=jnp.float32)

        @pl.when(j == 0)
        def _():
            acc_ref[sl, :] = part.astype(jnp.bfloat16)

        @pl.when(jnp.logical_and(j > 0, j < NF - 1))
        def _():
            acc_ref[sl, :] = (acc_ref[sl, :].astype(jnp.float32)
                              + part).astype(jnp.bfloat16)

        @pl.when(j == NF - 1)
        def _():
            ys_ref[...] = acc_ref[sl, :].astype(jnp.float32) + part


def _grouped_ffn(meta, xs, w1, w3, w2):
    grid_spec = pltpu.PrefetchScalarGridSpec(
        num_scalar_prefetch=1,
        grid=(NF, NB),
        in_specs=[
            pl.BlockSpec((BLK, D), lambda j, b, s: (b, 0)),
            pl.BlockSpec((1, D, FT), lambda j, b, s: (s[b], 0, j)),
            pl.BlockSpec((1, D, FT), lambda j, b, s: (s[b], 0, j)),
            pl.BlockSpec((1, FT, D), lambda j, b, s: (s[b], j, 0)),
        ],
        out_specs=pl.BlockSpec(
            (BLK, D), lambda j, b, s: (jnp.where(j == NF - 1, b, 0), 0)),
        scratch_shapes=[pltpu.VMEM((NP, D), jnp.bfloat16)],
    )
    return pl.pallas_call(
        _k3_body,
        grid_spec=grid_spec,
        out_shape=jax.ShapeDtypeStruct((NP, D), jnp.float32),
        interpret=_INTERPRET,
    )(meta, xs, w1, w3, w2)


def _k4_body(ys_hbm, pos0_hbm, pos1_hbm, w0_hbm, w1_hbm, out_hbm,
             idxa_v, idxb_v, bufa_v, bufb_v, wa_v, wb_v, sem):
    wid = lax.axis_index("s") * NC + lax.axis_index("c")
    hh = CH // 2
    for h in range(2):
        base = wid * CH + h * hh
        pltpu.sync_copy(pos0_hbm.at[pl.ds(base, hh)], idxa_v)
        pltpu.sync_copy(pos1_hbm.at[pl.ds(base, hh)], idxb_v)
        pltpu.sync_copy(w0_hbm.at[pl.ds(base, hh)], wa_v)
        pltpu.sync_copy(w1_hbm.at[pl.ds(base, hh)], wb_v)
        pltpu.async_copy(ys_hbm.at[idxa_v], bufa_v, sem).wait()
        pltpu.async_copy(ys_hbm.at[idxb_v], bufb_v, sem).wait()

        def addrow(r, carry):
            w0v = wa_v[r, 0:16]
            w1v = wb_v[r, 0:16]
            for cblk in range(D // 16):
                cs = pl.ds(cblk * 16, 16)
                bufa_v[r, cs] = bufa_v[r, cs] * w0v + bufb_v[r, cs] * w1v
            return carry

        lax.fori_loop(0, hh, addrow, 0)
        pltpu.sync_copy(bufa_v, out_hbm.at[pl.ds(base, hh)])


def _sc_combine(ys, pos0, pos1, w0o, w1o):
    mesh = plsc.VectorSubcoreMesh(core_axis_name="c", subcore_axis_name="s",
                                  num_cores=NC, num_subcores=NS)
    fn = functools.partial(
        pl.kernel,
        out_type=jax.ShapeDtypeStruct((T, D), jnp.float32),
        mesh=mesh,
        scratch_types=[
            pltpu.VMEM((CH // 2,), jnp.int32),
            pltpu.VMEM((CH // 2,), jnp.int32),
            pltpu.VMEM((CH // 2, D), jnp.float32),
            pltpu.VMEM((CH // 2, D), jnp.float32),
            pltpu.VMEM((CH // 2, RWL), jnp.float32),
            pltpu.VMEM((CH // 2, RWL), jnp.float32),
            pltpu.SemaphoreType.DMA,
        ],
        interpret=_INTERPRET,
    )(_k4_body)
    return fn(ys, pos0, pos1, w0o, w1o)


def kernel(hidden_states, w_router, w1, w2, w3):
    shape = hidden_states.shape
    x2d = hidden_states.reshape(T, D)
    pos0, pos1, w0o, w1o, meta = _router_dispatch(x2d, w_router)
    xs = _sc_scatter(x2d, pos0, pos1)
    ys = _grouped_ffn(meta, xs, w1, w3, w2)
    out2d = _sc_combine(ys, pos0, pos1, w0o, w1o)
    return (out2d.reshape(shape), jnp.zeros((1,), jnp.float32))


# speedup vs baseline: 1.1455x; 1.0043x over previous
import functools

import jax
import jax.numpy as jnp
from jax import lax
from jax.experimental import pallas as pl
from jax.experimental.pallas import tpu as pltpu
from jax.experimental.pallas import tpu_sc as plsc

E = 8
TOPK = 2
D = 1024
F = 4096
T = 2048

BLK = 256
NP = TOPK * T + E * BLK
NB = NP // BLK
FT = 1024
NF = F // FT

NC = 2
NS = 16
NW = NC * NS
CH = T // NW
RWL = 128

_INTERPRET = False


def _k1_body(x_ref, wr_ref, pos0_ref, pos1_ref, w0_ref, w1_ref, meta_ref):
    x = x_ref[...]
    xb = x.astype(jnp.bfloat16).astype(jnp.float32)
    logits = jnp.dot(xb, wr_ref[...], preferred_element_type=jnp.float32)
    m = jnp.max(logits, axis=1, keepdims=True)
    ex = jnp.exp(logits - m)
    p = ex / jnp.sum(ex, axis=1, keepdims=True)

    lane = lax.broadcasted_iota(jnp.int32, (T, E), 1)
    v0 = jnp.max(p, axis=1, keepdims=True)
    i0 = jnp.min(jnp.where(p == v0, lane, E), axis=1, keepdims=True)
    p1 = jnp.where(lane == i0, -jnp.inf, p)
    v1 = jnp.max(p1, axis=1, keepdims=True)
    i1 = jnp.min(jnp.where(p1 == v1, lane, E), axis=1, keepdims=True)
    s = v0 + v1
    w0n = v0 / s
    w1n = v1 / s

    a = (lane == i0).astype(jnp.int32) + (lane == i1).astype(jnp.int32)
    c = a
    sft = 1
    while sft < T:
        shifted = jnp.concatenate(
            [jnp.zeros((sft, E), jnp.int32), c[:T - sft]], axis=0)
        c = c + shifted
        sft *= 2
    p_exc = c - a
    counts = c[T - 1:T, :]
    padded = ((counts + (BLK - 1)) >> 8) << 8

    l8 = lax.broadcasted_iota(jnp.int32, (1, E), 1)
    st = padded
    sft = 1
    while sft < E:
        st = st + jnp.where(l8 >= sft, pltpu.roll(st, sft, axis=1), 0)
        sft *= 2
    starts = st - padded

    tot_f = (starts + p_exc).astype(jnp.float32)
    posv0 = jnp.sum(jnp.where(lane == i0, tot_f, 0.0), axis=1, keepdims=True)
    posv1 = jnp.sum(jnp.where(lane == i1, tot_f, 0.0), axis=1, keepdims=True)

    A = (jnp.where(lane == 0, posv0, 0.0) + jnp.where(lane == 1, posv1, 0.0)
         + jnp.where(lane == 2, w0n, 0.0) + jnp.where(lane == 3, w1n, 0.0))
    ii = lax.broadcasted_iota(jnp.int32, (E, E), 0)
    jj = lax.broadcasted_iota(jnp.int32, (E, E), 1)
    i8 = (ii == jj).astype(jnp.float32)
    at = lax.dot_general(i8, A, (((1,), (1,)), ((), ())),
                         preferred_element_type=jnp.float32,
                         precision=lax.Precision.HIGHEST)
    pos0_ref[...] = at[0:1].reshape(T).astype(jnp.int32)
    pos1_ref[...] = at[1:2].reshape(T).astype(jnp.int32)
    w0_ref[...] = jnp.broadcast_to(w0n, (T, RWL))
    w1_ref[...] = jnp.broadcast_to(w1n, (T, RWL))

    nactb = jnp.sum(padded) >> 8
    le = jnp.max(jnp.where(padded > 0, l8, -1))
    startsT = lax.dot_general(i8, starts.astype(jnp.float32),
                              (((1,), (1,)), ((), ())),
                              preferred_element_type=jnp.float32,
                              precision=lax.Precision.HIGHEST)
    paddedT = lax.dot_general(i8, padded.astype(jnp.float32),
                              (((1,), (1,)), ((), ())),
                              preferred_element_type=jnp.float32,
                              precision=lax.Precision.HIGHEST)
    bB = lax.broadcasted_iota(jnp.int32, (E, 128), 1).astype(
        jnp.float32) * float(BLK)
    in_e = jnp.logical_and(bB >= startsT, bB < startsT + paddedT)
    rowe = lax.broadcasted_iota(jnp.int32, (E, 128), 0).astype(jnp.float32)
    be = jnp.sum(rowe * in_e.astype(jnp.float32), axis=0,
                 keepdims=True).astype(jnp.int32)
    bi = lax.broadcasted_iota(jnp.int32, (1, 128), 1)
    meta = jnp.where(bi == 100, nactb, jnp.where(bi < nactb, be, le))
    meta_ref[...] = meta.reshape(128)


def _router_dispatch(x2d, w_router):
    return pl.pallas_call(
        _k1_body,
        out_shape=[
            jax.ShapeDtypeStruct((T,), jnp.int32),
            jax.ShapeDtypeStruct((T,), jnp.int32),
            jax.ShapeDtypeStruct((T, RWL), jnp.float32),
            jax.ShapeDtypeStruct((T, RWL), jnp.float32),
            jax.ShapeDtypeStruct((128,), jnp.int32),
        ],
        interpret=_INTERPRET,
    )(x2d, w_router)


def _k2_body(x_hbm, pos0_hbm, pos1_hbm, xs_hbm, idx0_v, idx1_v, xrows_v, sem):
    wid = lax.axis_index("s") * NC + lax.axis_index("c")
    base = wid * CH
    pltpu.sync_copy(pos0_hbm.at[pl.ds(base, CH)], idx0_v)
    pltpu.sync_copy(pos1_hbm.at[pl.ds(base, CH)], idx1_v)
    pltpu.sync_copy(x_hbm.at[pl.ds(base, CH)], xrows_v)
    cp0 = pltpu.async_copy(xrows_v, xs_hbm.at[idx0_v], sem)
    cp1 = pltpu.async_copy(xrows_v, xs_hbm.at[idx1_v], sem)
    cp0.wait()
    cp1.wait()


def _sc_scatter(x2d, pos0, pos1):
    mesh = plsc.VectorSubcoreMesh(core_axis_name="c", subcore_axis_name="s",
                                  num_cores=NC, num_subcores=NS)
    fn = functools.partial(
        pl.kernel,
        out_type=jax.ShapeDtypeStruct((NP, D), jnp.float32),
        mesh=mesh,
        scratch_types=[
            pltpu.VMEM((CH,), jnp.int32),
            pltpu.VMEM((CH,), jnp.int32),
            pltpu.VMEM((CH, D), jnp.float32),
            pltpu.SemaphoreType.DMA,
        ],
        interpret=_INTERPRET,
    )(_k2_body)
    return fn(x2d, pos0, pos1)


def _k3_body(s_ref, xs_ref, w1_ref, w3_ref, w2_ref, ys_ref, acc_ref):
    j = pl.program_id(0)
    b = pl.program_id(1)
    nact = s_ref[100]

    @pl.when(b < nact)
    def _():
        sl = pl.ds(b * BLK, BLK)
        x = xs_ref[...]
        h1 = jnp.dot(x, w1_ref[0], preferred_element_type=jnp.float32)
        h3 = jnp.dot(x, w3_ref[0], preferred_element_type=jnp.float32)
        h = h1 * (1.0 / (1.0 + jnp.exp(-h1))) * h3
        part = jnp.dot(h, w2_ref[0], preferred_element_type=jnp.float32)

        @pl.when(j == 0)
        def _():
            acc_ref[sl, :] = part.astype(jnp.bfloat16)

        @pl.when(jnp.logical_and(j > 0, j < NF - 1))
        def _():
            acc_ref[sl, :] = (acc_ref[sl, :].astype(jnp.float32)
                              + part).astype(jnp.bfloat16)

        @pl.when(j == NF - 1)
        def _():
            ys_ref[...] = acc_ref[sl, :].astype(jnp.float32) + part


def _grouped_ffn(meta, xs, w1, w3, w2):
    grid_spec = pltpu.PrefetchScalarGridSpec(
        num_scalar_prefetch=1,
        grid=(NF, NB),
        in_specs=[
            pl.BlockSpec((BLK, D), lambda j, b, s: (b, 0)),
            pl.BlockSpec((1, D, FT), lambda j, b, s: (s[b], 0, j)),
            pl.BlockSpec((1, D, FT), lambda j, b, s: (s[b], 0, j)),
            pl.BlockSpec((1, FT, D), lambda j, b, s: (s[b], j, 0)),
        ],
        out_specs=pl.BlockSpec(
            (BLK, D), lambda j, b, s: (jnp.where(j == NF - 1, b, 0), 0)),
        scratch_shapes=[pltpu.VMEM((NP, D), jnp.bfloat16)],
    )
    return pl.pallas_call(
        _k3_body,
        grid_spec=grid_spec,
        out_shape=jax.ShapeDtypeStruct((NP, D), jnp.float32),
        interpret=_INTERPRET,
    )(meta, xs, w1, w3, w2)


def _k4_body(ys_hbm, pos0_hbm, pos1_hbm, w0_hbm, w1_hbm, out_hbm,
             idxa_v, idxb_v, bufa_v, bufb_v, wa_v, wb_v, sem):
    wid = lax.axis_index("s") * NC + lax.axis_index("c")
    hh = CH // 2
    for h in range(2):
        base = wid * CH + h * hh
        pltpu.sync_copy(pos0_hbm.at[pl.ds(base, hh)], idxa_v)
        pltpu.sync_copy(pos1_hbm.at[pl.ds(base, hh)], idxb_v)
        pltpu.sync_copy(w0_hbm.at[pl.ds(base, hh)], wa_v)
        pltpu.sync_copy(w1_hbm.at[pl.ds(base, hh)], wb_v)
        cpa = pltpu.async_copy(ys_hbm.at[idxa_v], bufa_v, sem)
        cpb = pltpu.async_copy(ys_hbm.at[idxb_v], bufb_v, sem)
        cpa.wait()
        cpb.wait()

        def addrow(r, carry):
            w0v = wa_v[r, 0:16]
            w1v = wb_v[r, 0:16]
            for cblk in range(D // 16):
                cs = pl.ds(cblk * 16, 16)
                bufa_v[r, cs] = bufa_v[r, cs] * w0v + bufb_v[r, cs] * w1v
            return carry

        lax.fori_loop(0, hh, addrow, 0)
        pltpu.sync_copy(bufa_v, out_hbm.at[pl.ds(base, hh)])


def _sc_combine(ys, pos0, pos1, w0o, w1o):
    mesh = plsc.VectorSubcoreMesh(core_axis_name="c", subcore_axis_name="s",
                                  num_cores=NC, num_subcores=NS)
    fn = functools.partial(
        pl.kernel,
        out_type=jax.ShapeDtypeStruct((T, D), jnp.float32),
        mesh=mesh,
        scratch_types=[
            pltpu.VMEM((CH // 2,), jnp.int32),
            pltpu.VMEM((CH // 2,), jnp.int32),
            pltpu.VMEM((CH // 2, D), jnp.float32),
            pltpu.VMEM((CH // 2, D), jnp.float32),
            pltpu.VMEM((CH // 2, RWL), jnp.float32),
            pltpu.VMEM((CH // 2, RWL), jnp.float32),
            pltpu.SemaphoreType.DMA,
        ],
        interpret=_INTERPRET,
    )(_k4_body)
    return fn(ys, pos0, pos1, w0o, w1o)


def kernel(hidden_states, w_router, w1, w2, w3):
    shape = hidden_states.shape
    x2d = hidden_states.reshape(T, D)
    pos0, pos1, w0o, w1o, meta = _router_dispatch(x2d, w_router)
    xs = _sc_scatter(x2d, pos0, pos1)
    ys = _grouped_ffn(meta, xs, w1, w3, w2)
    out2d = _sc_combine(ys, pos0, pos1, w0o, w1o)
    return (out2d.reshape(shape), jnp.zeros((1,), jnp.float32))


# speedup vs baseline: 1.2629x; 1.1025x over previous
import functools

import jax
import jax.numpy as jnp
from jax import lax
from jax.experimental import pallas as pl
from jax.experimental.pallas import tpu as pltpu
from jax.experimental.pallas import tpu_sc as plsc

E = 8
TOPK = 2
D = 1024
F = 4096
T = 2048

BLK = 512
NP = TOPK * T + E * BLK
NB = NP // BLK
FT = 1024
NF = F // FT

NC = 2
NS = 16
NW = NC * NS
CH = T // NW
RWL = 128
SB = BLK.bit_length() - 1

_INTERPRET = False


def _k1_body(x_ref, wr_ref, pos0_ref, pos1_ref, w0_ref, w1_ref, meta_ref):
    x = x_ref[...]
    xb = x.astype(jnp.bfloat16).astype(jnp.float32)
    logits = jnp.dot(xb, wr_ref[...], preferred_element_type=jnp.float32)
    m = jnp.max(logits, axis=1, keepdims=True)
    ex = jnp.exp(logits - m)
    p = ex / jnp.sum(ex, axis=1, keepdims=True)

    lane = lax.broadcasted_iota(jnp.int32, (T, E), 1)
    v0 = jnp.max(p, axis=1, keepdims=True)
    i0 = jnp.min(jnp.where(p == v0, lane, E), axis=1, keepdims=True)
    p1 = jnp.where(lane == i0, -jnp.inf, p)
    v1 = jnp.max(p1, axis=1, keepdims=True)
    i1 = jnp.min(jnp.where(p1 == v1, lane, E), axis=1, keepdims=True)
    s = v0 + v1
    w0n = v0 / s
    w1n = v1 / s

    a = (lane == i0).astype(jnp.int32) + (lane == i1).astype(jnp.int32)
    c = a
    sft = 1
    while sft < T:
        shifted = jnp.concatenate(
            [jnp.zeros((sft, E), jnp.int32), c[:T - sft]], axis=0)
        c = c + shifted
        sft *= 2
    p_exc = c - a
    counts = c[T - 1:T, :]
    padded = ((counts + (BLK - 1)) >> SB) << SB

    l8 = lax.broadcasted_iota(jnp.int32, (1, E), 1)
    st = padded
    sft = 1
    while sft < E:
        st = st + jnp.where(l8 >= sft, pltpu.roll(st, sft, axis=1), 0)
        sft *= 2
    starts = st - padded

    tot_f = (starts + p_exc).astype(jnp.float32)
    posv0 = jnp.sum(jnp.where(lane == i0, tot_f, 0.0), axis=1, keepdims=True)
    posv1 = jnp.sum(jnp.where(lane == i1, tot_f, 0.0), axis=1, keepdims=True)

    A = (jnp.where(lane == 0, posv0, 0.0) + jnp.where(lane == 1, posv1, 0.0)
         + jnp.where(lane == 2, w0n, 0.0) + jnp.where(lane == 3, w1n, 0.0))
    ii = lax.broadcasted_iota(jnp.int32, (E, E), 0)
    jj = lax.broadcasted_iota(jnp.int32, (E, E), 1)
    i8 = (ii == jj).astype(jnp.float32)
    at = lax.dot_general(i8, A, (((1,), (1,)), ((), ())),
                         preferred_element_type=jnp.float32,
                         precision=lax.Precision.HIGHEST)
    pos0_ref[...] = at[0:1].reshape(T).astype(jnp.int32)
    pos1_ref[...] = at[1:2].reshape(T).astype(jnp.int32)
    w0_ref[...] = jnp.broadcast_to(w0n, (T, RWL))
    w1_ref[...] = jnp.broadcast_to(w1n, (T, RWL))

    nactb = jnp.sum(padded) >> SB
    le = jnp.max(jnp.where(padded > 0, l8, -1))
    startsT = lax.dot_general(i8, starts.astype(jnp.float32),
                              (((1,), (1,)), ((), ())),
                              preferred_element_type=jnp.float32,
                              precision=lax.Precision.HIGHEST)
    paddedT = lax.dot_general(i8, padded.astype(jnp.float32),
                              (((1,), (1,)), ((), ())),
                              preferred_element_type=jnp.float32,
                              precision=lax.Precision.HIGHEST)
    bB = lax.broadcasted_iota(jnp.int32, (E, 128), 1).astype(
        jnp.float32) * float(BLK)
    in_e = jnp.logical_and(bB >= startsT, bB < startsT + paddedT)
    rowe = lax.broadcasted_iota(jnp.int32, (E, 128), 0).astype(jnp.float32)
    be = jnp.sum(rowe * in_e.astype(jnp.float32), axis=0,
                 keepdims=True).astype(jnp.int32)
    bi = lax.broadcasted_iota(jnp.int32, (1, 128), 1)
    meta = jnp.where(bi == 100, nactb, jnp.where(bi < nactb, be, le))
    meta_ref[...] = meta.reshape(128)


def _router_dispatch(x2d, w_router):
    return pl.pallas_call(
        _k1_body,
        out_shape=[
            jax.ShapeDtypeStruct((T,), jnp.int32),
            jax.ShapeDtypeStruct((T,), jnp.int32),
            jax.ShapeDtypeStruct((T, RWL), jnp.float32),
            jax.ShapeDtypeStruct((T, RWL), jnp.float32),
            jax.ShapeDtypeStruct((128,), jnp.int32),
        ],
        interpret=_INTERPRET,
    )(x2d, w_router)


def _k2_body(x_hbm, pos0_hbm, pos1_hbm, xs_hbm, idx0_v, idx1_v, xrows_v, sem):
    wid = lax.axis_index("s") * NC + lax.axis_index("c")
    base = wid * CH
    pltpu.sync_copy(pos0_hbm.at[pl.ds(base, CH)], idx0_v)
    pltpu.sync_copy(pos1_hbm.at[pl.ds(base, CH)], idx1_v)
    pltpu.sync_copy(x_hbm.at[pl.ds(base, CH)], xrows_v)
    cp0 = pltpu.async_copy(xrows_v, xs_hbm.at[idx0_v], sem)
    cp1 = pltpu.async_copy(xrows_v, xs_hbm.at[idx1_v], sem)
    cp0.wait()
    cp1.wait()


def _sc_scatter(x2d, pos0, pos1):
    mesh = plsc.VectorSubcoreMesh(core_axis_name="c", subcore_axis_name="s",
                                  num_cores=NC, num_subcores=NS)
    fn = functools.partial(
        pl.kernel,
        out_type=jax.ShapeDtypeStruct((NP, D), jnp.float32),
        mesh=mesh,
        scratch_types=[
            pltpu.VMEM((CH,), jnp.int32),
            pltpu.VMEM((CH,), jnp.int32),
            pltpu.VMEM((CH, D), jnp.float32),
            pltpu.SemaphoreType.DMA,
        ],
        interpret=_INTERPRET,
    )(_k2_body)
    return fn(x2d, pos0, pos1)


def _k3_body(s_ref, xs_ref, w1_ref, w3_ref, w2_ref, ys_ref, acc_ref):
    j = pl.program_id(0)
    b = pl.program_id(1)
    nact = s_ref[100]

    @pl.when(b < nact)
    def _():
        sl = pl.ds(b * BLK, BLK)
        x = xs_ref[...]
        h1 = jnp.dot(x, w1_ref[0], preferred_element_type=jnp.float32)
        h3 = jnp.dot(x, w3_ref[0], preferred_element_type=jnp.float32)
        h = h1 * (1.0 / (1.0 + jnp.exp(-h1))) * h3
        part = jnp.dot(h, w2_ref[0], preferred_element_type=jnp.float32)

        @pl.when(j == 0)
        def _():
            acc_ref[sl, :] = part.astype(jnp.bfloat16)

        @pl.when(jnp.logical_and(j > 0, j < NF - 1))
        def _():
            acc_ref[sl, :] = (acc_ref[sl, :].astype(jnp.float32)
                              + part).astype(jnp.bfloat16)

        @pl.when(j == NF - 1)
        def _():
            ys_ref[...] = acc_ref[sl, :].astype(jnp.float32) + part


def _grouped_ffn(meta, xs, w1, w3, w2):
    grid_spec = pltpu.PrefetchScalarGridSpec(
        num_scalar_prefetch=1,
        grid=(NF, NB),
        in_specs=[
            pl.BlockSpec((BLK, D), lambda j, b, s: (b, 0)),
            pl.BlockSpec((1, D, FT), lambda j, b, s: (s[b], 0, j)),
            pl.BlockSpec((1, D, FT), lambda j, b, s: (s[b], 0, j)),
            pl.BlockSpec((1, FT, D), lambda j, b, s: (s[b], j, 0)),
        ],
        out_specs=pl.BlockSpec(
            (BLK, D), lambda j, b, s: (jnp.where(j == NF - 1, b, 0), 0)),
        scratch_shapes=[pltpu.VMEM((NP, D), jnp.bfloat16)],
    )
    return pl.pallas_call(
        _k3_body,
        grid_spec=grid_spec,
        out_shape=jax.ShapeDtypeStruct((NP, D), jnp.float32),
        interpret=_INTERPRET,
    )(meta, xs, w1, w3, w2)


def _k4_body(ys_hbm, pos0_hbm, pos1_hbm, w0_hbm, w1_hbm, out_hbm,
             idxa_v, idxb_v, bufa_v, bufb_v, wa_v, wb_v, sem):
    wid = lax.axis_index("s") * NC + lax.axis_index("c")
    hh = CH // 2
    for h in range(2):
        base = wid * CH + h * hh
        pltpu.sync_copy(pos0_hbm.at[pl.ds(base, hh)], idxa_v)
        pltpu.sync_copy(pos1_hbm.at[pl.ds(base, hh)], idxb_v)
        pltpu.sync_copy(w0_hbm.at[pl.ds(base, hh)], wa_v)
        pltpu.sync_copy(w1_hbm.at[pl.ds(base, hh)], wb_v)
        cpa = pltpu.async_copy(ys_hbm.at[idxa_v], bufa_v, sem)
        cpb = pltpu.async_copy(ys_hbm.at[idxb_v], bufb_v, sem)
        cpa.wait()
        cpb.wait()

        def addrow(r, carry):
            w0v = wa_v[r, 0:16]
            w1v = wb_v[r, 0:16]
            for cblk in range(D // 16):
                cs = pl.ds(cblk * 16, 16)
                bufa_v[r, cs] = bufa_v[r, cs] * w0v + bufb_v[r, cs] * w1v
            return carry

        lax.fori_loop(0, hh, addrow, 0)
        pltpu.sync_copy(bufa_v, out_hbm.at[pl.ds(base, hh)])


def _sc_combine(ys, pos0, pos1, w0o, w1o):
    mesh = plsc.VectorSubcoreMesh(core_axis_name="c", subcore_axis_name="s",
                                  num_cores=NC, num_subcores=NS)
    fn = functools.partial(
        pl.kernel,
        out_type=jax.ShapeDtypeStruct((T, D), jnp.float32),
        mesh=mesh,
        scratch_types=[
            pltpu.VMEM((CH // 2,), jnp.int32),
            pltpu.VMEM((CH // 2,), jnp.int32),
            pltpu.VMEM((CH // 2, D), jnp.float32),
            pltpu.VMEM((CH // 2, D), jnp.float32),
            pltpu.VMEM((CH // 2, RWL), jnp.float32),
            pltpu.VMEM((CH // 2, RWL), jnp.float32),
            pltpu.SemaphoreType.DMA,
        ],
        interpret=_INTERPRET,
    )(_k4_body)
    return fn(ys, pos0, pos1, w0o, w1o)


def kernel(hidden_states, w_router, w1, w2, w3):
    shape = hidden_states.shape
    x2d = hidden_states.reshape(T, D)
    pos0, pos1, w0o, w1o, meta = _router_dispatch(x2d, w_router)
    xs = _sc_scatter(x2d, pos0, pos1)
    ys = _grouped_ffn(meta, xs, w1, w3, w2)
    out2d = _sc_combine(ys, pos0, pos1, w0o, w1o)
    return (out2d.reshape(shape), jnp.zeros((1,), jnp.float32))
